# baseline (speedup 1.0000x reference)
import functools

import jax
import jax.numpy as jnp
import numpy as np
from jax import lax
from jax.experimental import pallas as pl
from jax.experimental.pallas import tpu as pltpu
from jax.experimental.pallas import tpu_sc as plsc

_NUM_CATEGORIES = [100000] * 26
_F = len(_NUM_CATEGORIES)
_D = 32
_B = 16384
_BF = _B * _F

_info = plsc.get_sparse_core_info()
_NC, _NS = _info.num_cores, _info.num_subcores
_NW = _NC * _NS
_RPW = _BF // _NW
_PR = 128
_NP = _RPW // _PR

_offsets_np = np.cumsum([0] + _NUM_CATEGORIES[:-1]).astype(np.int32)
_OFF2 = np.concatenate([_offsets_np, _offsets_np])

_BIAS_PAT = _F * _D


def _sc_body(x_hbm, off2_hbm, bias_hbm, table_hbm, out_hbm,
             idx_all, rows, off2_v, bias2_v, sem):
    wid = lax.axis_index("s") * _NC + lax.axis_index("c")

    pltpu.sync_copy(off2_hbm, off2_v)
    pltpu.sync_copy(bias_hbm, bias2_v.at[pl.ds(0, _BIAS_PAT)])
    pltpu.sync_copy(bias_hbm, bias2_v.at[pl.ds(_BIAS_PAT, _BIAS_PAT)])

    row0 = wid * (_RPW // _PR)
    pltpu.sync_copy(x_hbm.at[pl.ds(row0, _NP), :], idx_all)

    def off_row(r, q):
        for c in range(_PR // 16):
            chunk = idx_all[r, pl.ds(c * 16, 16)]
            idx_all[r, pl.ds(c * 16, 16)] = chunk + off2_v[pl.ds(q, 16)]
            q = q + 16
            q = jnp.where(q >= _F, q - _F, q)
        return q

    lax.fori_loop(0, _NP, off_row, jnp.int32(0), unroll=False)

    base = wid * _RPW

    def piece(i, carry):
        pltpu.async_copy(table_hbm.at[idx_all.at[i]], rows, sem).wait()

        qb0 = lax.rem(i * _PR, _F) * _D

        def bias_row(r, qb):
            rows[r, pl.ds(0, 16)] = rows[r, pl.ds(0, 16)] + bias2_v[pl.ds(qb, 16)]
            rows[r, pl.ds(16, 16)] = rows[r, pl.ds(16, 16)] + bias2_v[pl.ds(qb + 16, 16)]
            qb = qb + _D
            return jnp.where(qb >= _BIAS_PAT, qb - _BIAS_PAT, qb)

        lax.fori_loop(0, _PR, bias_row, qb0, unroll=False)

        pltpu.sync_copy(rows, out_hbm.at[pl.ds(base + i * _PR, _PR), :])
        return carry

    lax.fori_loop(0, _NP, piece, jnp.int32(0), unroll=False)


@jax.jit
def _tokenize(x2d, off2, bias_flat, table):
    mesh = plsc.VectorSubcoreMesh(core_axis_name="c", subcore_axis_name="s")
    kern = functools.partial(
        pl.kernel,
        mesh=mesh,
        out_type=jax.ShapeDtypeStruct((_BF, _D), jnp.float32),
        scratch_types=[
            pltpu.VMEM((_NP, _PR), jnp.int32),
            pltpu.VMEM((_PR, _D), jnp.float32),
            pltpu.VMEM((2 * _F,), jnp.int32),
            pltpu.VMEM((2 * _BIAS_PAT,), jnp.float32),
            pltpu.SemaphoreType.DMA,
        ],
        compiler_params=pltpu.CompilerParams(use_tc_tiling_on_sc=False),
    )(_sc_body)
    return kern(x2d, off2, bias_flat, table)


def kernel(x, table, bias):
    x2d = x.reshape(_BF // _PR, _PR)
    out = _tokenize(x2d, jnp.asarray(_OFF2), bias.reshape(-1), table)
    return out.reshape(_B, _F, _D)
